# Initial kernel scaffold; baseline (speedup 1.0000x reference)
import jax
import jax.numpy as jnp
from jax.experimental import pallas as pl


def kernel(x, buf, ptr, src, dst, edge_weights, bias, Wq, Wk, Wv, Wg, bg, gamma, beta):
    raise NotImplementedError("write your pallas kernel here")


# speedup vs baseline: 61.4485x; 61.4485x over previous
import functools

import jax
import jax.numpy as jnp
from jax import lax
from jax.experimental import pallas as pl
from jax.experimental.pallas import tpu as pltpu
from jax.experimental.pallas import tpu_sc as plsc

_D = 4096
_NC = 2
_NS = 16
_NW = _NC * _NS
_LANES = 16


def _edge_scatter_partials(x1d, src, dst, w):
    E = src.shape[0]
    epw = E // _NW
    mesh = plsc.VectorSubcoreMesh(core_axis_name="c", subcore_axis_name="s")

    @functools.partial(
        pl.kernel,
        mesh=mesh,
        out_type=jax.ShapeDtypeStruct((_NW, _D), jnp.float32),
        compiler_params=pltpu.CompilerParams(needs_layout_passes=False),
        scratch_types=[
            pltpu.VMEM((_D,), jnp.float32),
            pltpu.VMEM((_D,), jnp.float32),
            pltpu.VMEM((epw,), jnp.int32),
            pltpu.VMEM((epw,), jnp.int32),
            pltpu.VMEM((epw,), jnp.float32),
        ],
    )
    def k(x_hbm, src_hbm, dst_hbm, w_hbm, out_hbm, x_v, acc_v, src_v, dst_v, w_v):
        wid = lax.axis_index("s") * _NC + lax.axis_index("c")
        base = wid * epw
        pltpu.sync_copy(x_hbm, x_v)
        pltpu.sync_copy(src_hbm.at[pl.ds(base, epw)], src_v)
        pltpu.sync_copy(dst_hbm.at[pl.ds(base, epw)], dst_v)
        pltpu.sync_copy(w_hbm.at[pl.ds(base, epw)], w_v)

        def zero_body(i, carry):
            acc_v[pl.ds(i * _LANES, _LANES)] = jnp.zeros((_LANES,), jnp.float32)
            return carry

        lax.fori_loop(0, _D // _LANES, zero_body, 0)

        def edge_body(g, carry):
            o = g * _LANES
            idx = src_v[pl.ds(o, _LANES)]
            dd = dst_v[pl.ds(o, _LANES)]
            wt = w_v[pl.ds(o, _LANES)]
            vals = plsc.load_gather(x_v, [idx])
            plsc.addupdate_scatter(acc_v, [dd], vals * wt)
            return carry

        lax.fori_loop(0, epw // _LANES, edge_body, 0)
        pltpu.sync_copy(acc_v, out_hbm.at[wid])

    return k(x1d, src, dst, w)


def _matvec_rows(W, xrow, tile=256):
    N, K = W.shape

    def body(w_ref, x_ref, o_ref):
        o_ref[0, :] = jnp.sum(w_ref[...] * x_ref[0, :][None, :], axis=1)

    return pl.pallas_call(
        body,
        grid=(N // tile,),
        in_specs=[
            pl.BlockSpec((tile, K), lambda i: (i, 0)),
            pl.BlockSpec((1, K), lambda i: (0, 0)),
        ],
        out_specs=pl.BlockSpec((1, tile), lambda i: (0, i)),
        out_shape=jax.ShapeDtypeStruct((1, N), jnp.float32),
    )(W, xrow)


def _vecmat(qrow, W, tile=256):
    K, N = W.shape

    def body(q_ref, w_ref, o_ref):
        i = pl.program_id(0)

        @pl.when(i == 0)
        def _():
            o_ref[...] = jnp.zeros_like(o_ref)

        o_ref[0, :] += jnp.sum(w_ref[...] * q_ref[0, :][:, None], axis=0)

    return pl.pallas_call(
        body,
        grid=(K // tile,),
        in_specs=[
            pl.BlockSpec((1, tile), lambda i: (0, i)),
            pl.BlockSpec((tile, N), lambda i: (i, 0)),
        ],
        out_specs=pl.BlockSpec((1, N), lambda i: (0, 0)),
        out_shape=jax.ShapeDtypeStruct((1, N), jnp.float32),
    )(qrow, W)


def _flash_over_buf(buf, kqrow, scale, tile=256):
    Bn, Dn = buf.shape

    def body(buf_ref, kq_ref, nb_ref, c_ref, stat_ref, acc_ref):
        i = pl.program_id(0)
        b = buf_ref[...]
        nb_ref[...] = b
        s = jnp.sum(b * kq_ref[0, :][None, :], axis=1) * scale

        @pl.when(i == 0)
        def _():
            stat_ref[0, 0] = -jnp.inf
            stat_ref[0, 1] = 0.0
            acc_ref[...] = jnp.zeros_like(acc_ref)

        m_old = stat_ref[0, 0]
        l_old = stat_ref[0, 1]
        m_new = jnp.maximum(m_old, jnp.max(s))
        alpha = jnp.exp(m_old - m_new)
        e = jnp.exp(s - m_new)
        stat_ref[0, 0] = m_new
        stat_ref[0, 1] = l_old * alpha + jnp.sum(e)
        acc_ref[...] = acc_ref[...] * alpha + jnp.sum(e[:, None] * b, axis=0)[None, :]

        @pl.when(i == pl.num_programs(0) - 1)
        def _():
            c_ref[...] = acc_ref[...] / stat_ref[0, 1]

    return pl.pallas_call(
        body,
        grid=(Bn // tile,),
        in_specs=[
            pl.BlockSpec((tile, Dn), lambda i: (i, 0)),
            pl.BlockSpec((1, Dn), lambda i: (0, 0)),
        ],
        out_specs=[
            pl.BlockSpec((tile, Dn), lambda i: (i, 0)),
            pl.BlockSpec((1, Dn), lambda i: (0, 0)),
        ],
        out_shape=[
            jax.ShapeDtypeStruct((Bn, Dn), jnp.float32),
            jax.ShapeDtypeStruct((1, Dn), jnp.float32),
        ],
        scratch_shapes=[
            pltpu.SMEM((1, 2), jnp.float32),
            pltpu.VMEM((1, Dn), jnp.float32),
        ],
    )(buf, kqrow)


def _sparse_finish(partials, bias_row):
    def body(p_ref, b_ref, o_ref):
        s = jnp.sum(p_ref[...], axis=0) + b_ref[0, :]
        o_ref[0, :] = s * 0.5 * (1.0 + lax.erf(s * (2.0 ** -0.5)))

    return pl.pallas_call(
        body,
        grid=(1,),
        in_specs=[
            pl.BlockSpec((_NW, _D), lambda i: (0, 0)),
            pl.BlockSpec((1, _D), lambda i: (0, 0)),
        ],
        out_specs=pl.BlockSpec((1, _D), lambda i: (0, 0)),
        out_shape=jax.ShapeDtypeStruct((1, _D), jnp.float32),
    )(partials, bias_row)


def _gate_merge_norm(Wg, sp_row, rg_row, bg_row, x_row, gamma_row, beta_row, tile=256):
    Dn = sp_row.shape[1]

    def body(wg_ref, sp_ref, rg_ref, bg_ref, x_ref, gam_ref, bet_ref,
             o_ref, gated_ref):
        i = pl.program_id(0)
        z = (jnp.sum(wg_ref[:, :Dn] * sp_ref[0, :][None, :], axis=1)
             + jnp.sum(wg_ref[:, Dn:] * rg_ref[0, :][None, :], axis=1)
             + bg_ref[0, pl.ds(i * tile, tile)])
        g = jax.nn.sigmoid(z)
        sp_seg = sp_ref[0, pl.ds(i * tile, tile)]
        rg_seg = rg_ref[0, pl.ds(i * tile, tile)]
        gated_ref[0, pl.ds(i * tile, tile)] = g * sp_seg + (1.0 - g) * rg_seg

        @pl.when(i == pl.num_programs(0) - 1)
        def _():
            h = x_ref[0, :] + gated_ref[0, :]
            mu = jnp.mean(h)
            var = jnp.mean((h - mu) ** 2)
            o_ref[0, :] = ((h - mu) * lax.rsqrt(var + 1e-5) * gam_ref[0, :]
                           + bet_ref[0, :])

    return pl.pallas_call(
        body,
        grid=(Dn // tile,),
        in_specs=[
            pl.BlockSpec((tile, 2 * Dn), lambda i: (i, 0)),
            pl.BlockSpec((1, Dn), lambda i: (0, 0)),
            pl.BlockSpec((1, Dn), lambda i: (0, 0)),
            pl.BlockSpec((1, Dn), lambda i: (0, 0)),
            pl.BlockSpec((1, Dn), lambda i: (0, 0)),
            pl.BlockSpec((1, Dn), lambda i: (0, 0)),
            pl.BlockSpec((1, Dn), lambda i: (0, 0)),
        ],
        out_specs=pl.BlockSpec((1, Dn), lambda i: (0, 0)),
        out_shape=jax.ShapeDtypeStruct((1, Dn), jnp.float32),
        scratch_shapes=[pltpu.VMEM((1, Dn), jnp.float32)],
    )(Wg, sp_row, rg_row, bg_row, x_row, gamma_row, beta_row)


def kernel(x, buf, ptr, src, dst, edge_weights, bias, Wq, Wk, Wv, Wg, bg, gamma, beta):
    D = x.shape[0]
    Bn = buf.shape[0]
    x_row = x.reshape(1, D)

    partials = _edge_scatter_partials(x, src, dst, edge_weights)

    q_row = _matvec_rows(Wq, x_row)
    kq_row = _vecmat(q_row, Wk)

    nb_copy, chat_row = _flash_over_buf(buf, kq_row, D ** -0.5)

    ring_row = _matvec_rows(Wv, chat_row)

    sp_row = _sparse_finish(partials, bias.reshape(1, D))

    out_row = _gate_merge_norm(Wg, sp_row, ring_row, bg.reshape(1, D),
                               x_row, gamma.reshape(1, D), beta.reshape(1, D))

    out = out_row.reshape(D)
    r = (ptr % Bn).astype(jnp.int32)
    new_buf = lax.dynamic_update_slice(nb_copy, out_row, (r, 0))
    new_ptr = (ptr + 1) % Bn
    return out, new_buf, new_ptr
